# baseline (speedup 1.0000x reference)
import functools

import jax
import jax.numpy as jnp
from jax import lax
from jax.experimental import pallas as pl
from jax.experimental.pallas import tpu as pltpu
from jax.experimental.pallas import tpu_sc as plsc

SCALE = 8.0


@functools.cache
def _build(B, V, D):
    info = plsc.get_sparse_core_info()
    NC, NS, L = info.num_cores, info.num_subcores, info.num_lanes
    NW = NC * NS
    b_per_w = B // NW
    C = 640
    n_chunks = b_per_w // C
    assert b_per_w % C == 0 and D % L == 0

    mesh = plsc.VectorSubcoreMesh(core_axis_name="c", subcore_axis_name="s")

    @functools.partial(
        pl.kernel,
        mesh=mesh,
        out_type=jax.ShapeDtypeStruct((B, D), jnp.float32),
        scratch_types=[
            pltpu.VMEM((C,), jnp.int32),
            pltpu.VMEM((C, D), jnp.float32),
            pltpu.SemaphoreType.DMA,
        ],
        compiler_params=pltpu.CompilerParams(use_tc_tiling_on_sc=False),
    )
    def gather_scale(idx_hbm, table_hbm, out_hbm, idx_v, rows_v, sem):
        wid = lax.axis_index("s") * NC + lax.axis_index("c")
        base = wid * b_per_w

        def chunk_body(ci, carry):
            off = base + ci * C
            pltpu.sync_copy(idx_hbm.at[pl.ds(off, C)], idx_v)
            pltpu.async_copy(table_hbm.at[idx_v], rows_v, sem).wait()

            def row_body(r, c2):
                for j in range(D // L):
                    sl = pl.ds(j * L, L)
                    rows_v[r, sl] = rows_v[r, sl] * SCALE
                return c2

            lax.fori_loop(0, C, row_body, 0)
            pltpu.sync_copy(rows_v, out_hbm.at[pl.ds(off, C)])
            return carry

        lax.fori_loop(0, n_chunks, chunk_body, 0)

    return gather_scale


def kernel(x, lut):
    Bt, S = x.shape
    B = Bt * S
    xflat = x.reshape(B).astype(jnp.int32)
    out = _build(B, lut.shape[0], lut.shape[1])(xflat, lut)
    return out.reshape(Bt, S, lut.shape[1])


# speedup vs baseline: 1.0449x; 1.0449x over previous
import functools

import jax
import jax.numpy as jnp
from jax import lax
from jax.experimental import pallas as pl
from jax.experimental.pallas import tpu as pltpu
from jax.experimental.pallas import tpu_sc as plsc

SCALE = 8.0


@functools.cache
def _build(B, V, D):
    info = plsc.get_sparse_core_info()
    NC, NS, L = info.num_cores, info.num_subcores, info.num_lanes
    NW = NC * NS
    b_per_w = B // NW
    C = 640
    n_chunks = b_per_w // C
    U = 8
    assert b_per_w % C == 0 and D % L == 0 and C % U == 0

    mesh = plsc.VectorSubcoreMesh(core_axis_name="c", subcore_axis_name="s")

    @functools.partial(
        pl.kernel,
        mesh=mesh,
        out_type=jax.ShapeDtypeStruct((B, D), jnp.float32),
        scratch_types=[
            pltpu.VMEM((b_per_w,), jnp.int32),
            pltpu.VMEM((C, D), jnp.float32),
            pltpu.VMEM((C, D), jnp.float32),
            pltpu.SemaphoreType.DMA,
            pltpu.SemaphoreType.DMA,
            pltpu.SemaphoreType.DMA,
            pltpu.SemaphoreType.DMA,
        ],
        compiler_params=pltpu.CompilerParams(use_tc_tiling_on_sc=False),
    )
    def gather_scale(idx_hbm, table_hbm, out_hbm,
                     idx_v, rows0, rows1, g0, g1, o0, o1):
        wid = lax.axis_index("s") * NC + lax.axis_index("c")
        base = wid * b_per_w
        bufs = (rows0, rows1)
        gsems = (g0, g1)
        osems = (o0, o1)

        pltpu.sync_copy(idx_hbm.at[pl.ds(base, b_per_w)], idx_v)

        def scale(buf):
            def blk(rb, c):
                r0 = rb * U
                for u in range(U):
                    for j in range(D // L):
                        sl = pl.ds(j * L, L)
                        buf[r0 + u, sl] = buf[r0 + u, sl] * SCALE
                return c

            lax.fori_loop(0, C // U, blk, 0)

        gcopy = {}
        ocopy = {}

        def start_gather(ci, b):
            gcopy[b] = pltpu.async_copy(
                table_hbm.at[idx_v.at[pl.ds(ci * C, C)]], bufs[b], gsems[b])

        def start_out(ci, b):
            ocopy[b] = pltpu.async_copy(
                bufs[b], out_hbm.at[pl.ds(base + ci * C, C)], osems[b])

        start_gather(0, 0)
        for ci in range(n_chunks):
            b = ci % 2
            gcopy[b].wait()
            nxt = ci + 1
            if nxt < n_chunks:
                nb = nxt % 2
                if nxt >= 2:
                    ocopy[nb].wait()
                start_gather(nxt, nb)
            scale(bufs[b])
            start_out(ci, b)
        ocopy[0].wait()
        ocopy[1].wait()

    return gather_scale


def kernel(x, lut):
    Bt, S = x.shape
    B = Bt * S
    xflat = x.reshape(B).astype(jnp.int32)
    out = _build(B, lut.shape[0], lut.shape[1])(xflat, lut)
    return out.reshape(Bt, S, lut.shape[1])
